# baseline (speedup 1.0000x reference)
import functools

import jax
import jax.numpy as jnp
from jax import lax
from jax.experimental import pallas as pl
from jax.experimental.pallas import tpu as pltpu
from jax.experimental.pallas import tpu_sc as plsc

F32 = jnp.float32

_NC = 2
_NS = 16
_LANES = 16


def _mlp2_body(x_ref, w1_ref, b1_ref, w2_ref, b2_ref, o_ref):
    h = jnp.maximum(
        jnp.dot(x_ref[...], w1_ref[...], preferred_element_type=F32)
        + b1_ref[...], 0.0)
    o_ref[...] = jnp.dot(h, w2_ref[...], preferred_element_type=F32) + b2_ref[...]


def _mlp2(x, w1, b1, w2, b2, br):
    n, d = x.shape
    h = w1.shape[1]
    dout = w2.shape[1]
    return pl.pallas_call(
        _mlp2_body,
        grid=(n // br,),
        in_specs=[
            pl.BlockSpec((br, d), lambda i: (i, 0)),
            pl.BlockSpec((d, h), lambda i: (0, 0)),
            pl.BlockSpec((1, h), lambda i: (0, 0)),
            pl.BlockSpec((h, dout), lambda i: (0, 0)),
            pl.BlockSpec((1, dout), lambda i: (0, 0)),
        ],
        out_specs=pl.BlockSpec((br, dout), lambda i: (i, 0)),
        out_shape=jax.ShapeDtypeStruct((n, dout), F32),
    )(x, w1, b1.reshape(1, h), w2, b2.reshape(1, dout))


def _srcdst_body(x_ref, ws_ref, wd_ref, xs_ref, xd_ref):
    x = x_ref[...]
    xs_ref[...] = jnp.dot(x, ws_ref[...], preferred_element_type=F32)
    xd_ref[...] = jnp.dot(x, wd_ref[...], preferred_element_type=F32)


def _srcdst(x, ws, wd, br):
    n, h = x.shape
    return pl.pallas_call(
        _srcdst_body,
        grid=(n // br,),
        in_specs=[
            pl.BlockSpec((br, h), lambda i: (i, 0)),
            pl.BlockSpec((h, h), lambda i: (0, 0)),
            pl.BlockSpec((h, h), lambda i: (0, 0)),
        ],
        out_specs=[
            pl.BlockSpec((br, h), lambda i: (i, 0)),
            pl.BlockSpec((br, h), lambda i: (i, 0)),
        ],
        out_shape=[
            jax.ShapeDtypeStruct((n, h), F32),
            jax.ShapeDtypeStruct((n, h), F32),
        ],
    )(x, ws, wd)


def _eterm_body(ef_ref, w_ref, b_ref, o_ref):
    o_ref[...] = (
        jnp.dot(ef_ref[...], w_ref[...], preferred_element_type=F32)
        + b_ref[...]).astype(jnp.bfloat16)


def _eterm(ef, w, b, be_rows):
    e, de = ef.shape
    h = w.shape[1]
    return pl.pallas_call(
        _eterm_body,
        grid=(e // be_rows,),
        in_specs=[
            pl.BlockSpec((be_rows, de), lambda i: (i, 0)),
            pl.BlockSpec((de, h), lambda i: (0, 0)),
            pl.BlockSpec((1, h), lambda i: (0, 0)),
        ],
        out_specs=pl.BlockSpec((be_rows, h), lambda i: (i, 0)),
        out_shape=jax.ShapeDtypeStruct((e, h), jnp.bfloat16),
    )(ef, w, b.reshape(1, h))


def _update_body(x_ref, a0_ref, a1_ref, wn1_ref, wn2_ref, bn_ref, o_ref):
    x = x_ref[...]
    a = a0_ref[...] + a1_ref[...]
    u = (jnp.dot(x, wn1_ref[...], preferred_element_type=F32)
         + jnp.dot(a, wn2_ref[...], preferred_element_type=F32)
         + bn_ref[...])
    o_ref[...] = x + jnp.maximum(u, 0.0)


def _update(x, agg2, wn1, wn2, bn):
    n, h = x.shape
    npad = agg2.shape[0] // 2
    br = 640
    nb2 = npad // br
    return pl.pallas_call(
        _update_body,
        grid=(-(-n // br),),
        in_specs=[
            pl.BlockSpec((br, h), lambda i: (i, 0)),
            pl.BlockSpec((br, h), lambda i: (i, 0)),
            pl.BlockSpec((br, h), lambda i, nb2=nb2: (i + nb2, 0)),
            pl.BlockSpec((h, h), lambda i: (0, 0)),
            pl.BlockSpec((h, h), lambda i: (0, 0)),
            pl.BlockSpec((1, h), lambda i: (0, 0)),
        ],
        out_specs=pl.BlockSpec((br, h), lambda i: (i, 0)),
        out_shape=jax.ShapeDtypeStruct((n, h), F32),
    )(x, agg2, agg2, wn1, wn2, bn.reshape(1, h))


def _make_edge_pass(n, e, h):
    nw = _NC * _NS
    ew = e // nw
    k = 80
    nch = ew // k
    dr = 64
    npad = ((n + _NS * dr - 1) // (_NS * dr)) * (_NS * dr)
    rpt = npad // _NS
    ndr = rpt // dr
    hw = h // 32
    mesh = plsc.VectorSubcoreMesh(core_axis_name="c", subcore_axis_name="s")

    @functools.partial(
        pl.kernel,
        out_type=jax.ShapeDtypeStruct((2 * npad, h), F32),
        mesh=mesh,
        scratch_types=[
            pltpu.VMEM((k,), jnp.int32),
            pltpu.VMEM((k,), jnp.int32),
            pltpu.VMEM((k, h), F32),
            pltpu.VMEM((k, h), F32),
            pltpu.VMEM((k, h), jnp.bfloat16),
            pltpu.VMEM((k, h), F32),
            pltpu.VMEM((dr, h), F32),
            pltpu.VMEM_SHARED((npad, h), F32),
            pltpu.SemaphoreType.DMA,
            pltpu.SemaphoreType.DMA,
            pltpu.SemaphoreType.DMA,
        ],
    )
    def edge_pass(xs_hbm, xd_hbm, et_hbm, src_hbm, dst_hbm, out_hbm,
                  sidx, didx, xsb, xdb, etb, mb, stage, agg, g1, g2, g3):
        c = lax.axis_index("c")
        s = lax.axis_index("s")
        wid = s * _NC + c

        def zrow(i, carry):
            for j in range(h // _LANES):
                stage[i, pl.ds(j * _LANES, _LANES)] = jnp.zeros((_LANES,), F32)
            return carry
        lax.fori_loop(0, dr, zrow, 0)
        for q in range(ndr):
            pltpu.sync_copy(stage, agg.at[pl.ds(s * rpt + q * dr, dr)])
        plsc.subcore_barrier()

        def chunk(ci, carry):
            off = wid * ew + ci * k
            pltpu.sync_copy(src_hbm.at[pl.ds(off, k)], sidx)
            pltpu.sync_copy(dst_hbm.at[pl.ds(off, k)], didx)
            cp1 = pltpu.async_copy(xs_hbm.at[sidx], xsb, g1)
            cp2 = pltpu.async_copy(xd_hbm.at[didx], xdb, g2)
            cp3 = pltpu.async_copy(et_hbm.at[pl.ds(off, k)], etb, g3)
            cp1.wait()
            cp2.wait()
            cp3.wait()

            def vrow(i, inner):
                r0 = pl.multiple_of(i * 2, 2)
                for j in range(h // _LANES):
                    sl = pl.ds(j * _LANES, _LANES)
                    et2 = etb[pl.ds(r0, 2), sl].astype(F32)
                    mb[r0, sl] = jnp.maximum(
                        xsb[r0, sl] + xdb[r0, sl] + et2[0], 0.0)
                    mb[r0 + 1, sl] = jnp.maximum(
                        xsb[r0 + 1, sl] + xdb[r0 + 1, sl] + et2[1], 0.0)
                return inner
            lax.fori_loop(0, k // 2, vrow, 0)

            pltpu.sync_copy(mb, agg.at[didx], add=True)
            return carry
        lax.fori_loop(0, nch, chunk, 0)
        plsc.subcore_barrier()

        for q in range(ndr):
            pltpu.sync_copy(agg.at[pl.ds(s * rpt + q * dr, dr)], stage)
            pltpu.sync_copy(
                stage, out_hbm.at[pl.ds(c * npad + s * rpt + q * dr, dr)])

    return edge_pass


def kernel(node_features, edge_index, edge_features, enc_W1, enc_b1, enc_W2,
           enc_b2, We, be, Wn, bn, dec_W1, dec_b1, dec_W2, dec_b2):
    n, _ = node_features.shape
    e = edge_index.shape[1]
    h = enc_W1.shape[1]
    nlayers = We.shape[0]

    src = edge_index[0]
    dst = edge_index[1]

    br = 1000
    be_rows = 4000

    x = _mlp2(node_features, enc_W1, enc_b1, enc_W2, enc_b2, br)
    edge_pass = _make_edge_pass(n, e, h)

    for l in range(nlayers):
        ws = We[l, :h]
        wd = We[l, h:2 * h]
        wee = We[l, 2 * h:]
        xs, xd = _srcdst(x, ws, wd, br)
        et = _eterm(edge_features, wee, be[l], be_rows)
        agg2 = edge_pass(xs, xd, et, src, dst)
        x = _update(x, agg2, Wn[l, :h], Wn[l, h:], bn[l])

    return _mlp2(x, dec_W1, dec_b1, dec_W2, dec_b2, br)


# speedup vs baseline: 1.1034x; 1.1034x over previous
import functools

import jax
import jax.numpy as jnp
from jax import lax
from jax.experimental import pallas as pl
from jax.experimental.pallas import tpu as pltpu
from jax.experimental.pallas import tpu_sc as plsc

F32 = jnp.float32

_NC = 2
_NS = 16
_LANES = 16


def _mlp2_body(x_ref, w1_ref, b1_ref, w2_ref, b2_ref, o_ref):
    h = jnp.maximum(
        jnp.dot(x_ref[...], w1_ref[...], preferred_element_type=F32)
        + b1_ref[...], 0.0)
    o_ref[...] = jnp.dot(h, w2_ref[...], preferred_element_type=F32) + b2_ref[...]


def _mlp2(x, w1, b1, w2, b2, br):
    n, d = x.shape
    h = w1.shape[1]
    dout = w2.shape[1]
    return pl.pallas_call(
        _mlp2_body,
        grid=(n // br,),
        in_specs=[
            pl.BlockSpec((br, d), lambda i: (i, 0)),
            pl.BlockSpec((d, h), lambda i: (0, 0)),
            pl.BlockSpec((1, h), lambda i: (0, 0)),
            pl.BlockSpec((h, dout), lambda i: (0, 0)),
            pl.BlockSpec((1, dout), lambda i: (0, 0)),
        ],
        out_specs=pl.BlockSpec((br, dout), lambda i: (i, 0)),
        out_shape=jax.ShapeDtypeStruct((n, dout), F32),
    )(x, w1, b1.reshape(1, h), w2, b2.reshape(1, dout))


def _srcdst_body(x_ref, ws_ref, wd_ref, xs_ref, xd_ref):
    x = x_ref[...]
    xs_ref[...] = jnp.dot(x, ws_ref[...], preferred_element_type=F32)
    xd_ref[...] = jnp.dot(x, wd_ref[...], preferred_element_type=F32)


def _srcdst(x, ws, wd, br):
    n, h = x.shape
    return pl.pallas_call(
        _srcdst_body,
        grid=(n // br,),
        in_specs=[
            pl.BlockSpec((br, h), lambda i: (i, 0)),
            pl.BlockSpec((h, h), lambda i: (0, 0)),
            pl.BlockSpec((h, h), lambda i: (0, 0)),
        ],
        out_specs=[
            pl.BlockSpec((br, h), lambda i: (i, 0)),
            pl.BlockSpec((br, h), lambda i: (i, 0)),
        ],
        out_shape=[
            jax.ShapeDtypeStruct((n, h), F32),
            jax.ShapeDtypeStruct((n, h), F32),
        ],
    )(x, ws, wd)


def _eterm_body(ef_ref, w_ref, b_ref, o_ref):
    o_ref[...] = (
        jnp.dot(ef_ref[...], w_ref[...], preferred_element_type=F32)
        + b_ref[...])


def _eterm(ef, w, b, be_rows):
    e, de = ef.shape
    h = w.shape[1]
    return pl.pallas_call(
        _eterm_body,
        grid=(e // be_rows,),
        in_specs=[
            pl.BlockSpec((be_rows, de), lambda i: (i, 0)),
            pl.BlockSpec((de, h), lambda i: (0, 0)),
            pl.BlockSpec((1, h), lambda i: (0, 0)),
        ],
        out_specs=pl.BlockSpec((be_rows, h), lambda i: (i, 0)),
        out_shape=jax.ShapeDtypeStruct((e, h), F32),
    )(ef, w, b.reshape(1, h))


def _update_body(x_ref, a0_ref, a1_ref, wn1_ref, wn2_ref, bn_ref, o_ref):
    x = x_ref[...]
    a = a0_ref[...] + a1_ref[...]
    u = (jnp.dot(x, wn1_ref[...], preferred_element_type=F32)
         + jnp.dot(a, wn2_ref[...], preferred_element_type=F32)
         + bn_ref[...])
    o_ref[...] = x + jnp.maximum(u, 0.0)


def _update(x, agg2, wn1, wn2, bn):
    n, h = x.shape
    npad = agg2.shape[0] // 2
    br = 640
    nb2 = npad // br
    return pl.pallas_call(
        _update_body,
        grid=(-(-n // br),),
        in_specs=[
            pl.BlockSpec((br, h), lambda i: (i, 0)),
            pl.BlockSpec((br, h), lambda i: (i, 0)),
            pl.BlockSpec((br, h), lambda i, nb2=nb2: (i + nb2, 0)),
            pl.BlockSpec((h, h), lambda i: (0, 0)),
            pl.BlockSpec((h, h), lambda i: (0, 0)),
            pl.BlockSpec((1, h), lambda i: (0, 0)),
        ],
        out_specs=pl.BlockSpec((br, h), lambda i: (i, 0)),
        out_shape=jax.ShapeDtypeStruct((n, h), F32),
    )(x, agg2, agg2, wn1, wn2, bn.reshape(1, h))


def _make_edge_pass(n, e, h):
    nw = _NC * _NS
    ew = e // nw
    k = 80
    nch = ew // k
    dr = 32
    npad = ((n + _NS * dr - 1) // (_NS * dr)) * (_NS * dr)
    rpt = npad // _NS
    ndr = rpt // dr
    hw = h // 32
    mesh = plsc.VectorSubcoreMesh(core_axis_name="c", subcore_axis_name="s")

    @functools.partial(
        pl.kernel,
        out_type=jax.ShapeDtypeStruct((2 * npad, h), F32),
        mesh=mesh,
        scratch_types=[
            pltpu.VMEM((k,), jnp.int32),
            pltpu.VMEM((k,), jnp.int32),
            pltpu.VMEM((k, h), F32),
            pltpu.VMEM((k, h), F32),
            pltpu.VMEM((k, h), F32),
            pltpu.VMEM((k, h), F32),
            pltpu.VMEM((dr, h), F32),
            pltpu.VMEM_SHARED((npad, h), F32),
            pltpu.SemaphoreType.DMA,
            pltpu.SemaphoreType.DMA,
            pltpu.SemaphoreType.DMA,
        ],
    )
    def edge_pass(xs_hbm, xd_hbm, et_hbm, src_hbm, dst_hbm, out_hbm,
                  sidx, didx, xsb, xdb, etb, mb, stage, agg, g1, g2, g3):
        c = lax.axis_index("c")
        s = lax.axis_index("s")
        wid = s * _NC + c

        def zrow(i, carry):
            for j in range(h // _LANES):
                stage[i, pl.ds(j * _LANES, _LANES)] = jnp.zeros((_LANES,), F32)
            return carry
        lax.fori_loop(0, dr, zrow, 0)
        for q in range(ndr):
            pltpu.sync_copy(stage, agg.at[pl.ds(s * rpt + q * dr, dr)])
        plsc.subcore_barrier()

        def chunk(ci, carry):
            off = wid * ew + ci * k
            pltpu.sync_copy(src_hbm.at[pl.ds(off, k)], sidx)
            pltpu.sync_copy(dst_hbm.at[pl.ds(off, k)], didx)
            cp1 = pltpu.async_copy(xs_hbm.at[sidx], xsb, g1)
            cp2 = pltpu.async_copy(xd_hbm.at[didx], xdb, g2)
            cp3 = pltpu.async_copy(et_hbm.at[pl.ds(off, k)], etb, g3)
            cp1.wait()
            cp2.wait()
            cp3.wait()

            def vrow(r, inner):
                for j in range(h // _LANES):
                    sl = pl.ds(j * _LANES, _LANES)
                    mb[r, sl] = jnp.maximum(
                        xsb[r, sl] + xdb[r, sl] + etb[r, sl], 0.0)
                return inner
            lax.fori_loop(0, k, vrow, 0)

            pltpu.sync_copy(mb, agg.at[didx], add=True)
            return carry
        lax.fori_loop(0, nch, chunk, 0)
        plsc.subcore_barrier()

        for q in range(ndr):
            pltpu.sync_copy(agg.at[pl.ds(s * rpt + q * dr, dr)], stage)
            pltpu.sync_copy(
                stage, out_hbm.at[pl.ds(c * npad + s * rpt + q * dr, dr)])

    return edge_pass


def kernel(node_features, edge_index, edge_features, enc_W1, enc_b1, enc_W2,
           enc_b2, We, be, Wn, bn, dec_W1, dec_b1, dec_W2, dec_b2):
    n, _ = node_features.shape
    e = edge_index.shape[1]
    h = enc_W1.shape[1]
    nlayers = We.shape[0]

    src = edge_index[0]
    dst = edge_index[1]

    br = 1000
    be_rows = 4000

    x = _mlp2(node_features, enc_W1, enc_b1, enc_W2, enc_b2, br)
    edge_pass = _make_edge_pass(n, e, h)

    for l in range(nlayers):
        ws = We[l, :h]
        wd = We[l, h:2 * h]
        wee = We[l, 2 * h:]
        xs, xd = _srcdst(x, ws, wd, br)
        et = _eterm(edge_features, wee, be[l], be_rows)
        agg2 = edge_pass(xs, xd, et, src, dst)
        x = _update(x, agg2, Wn[l, :h], Wn[l, h:], bn[l])

    return _mlp2(x, dec_W1, dec_b1, dec_W2, dec_b2, br)


# speedup vs baseline: 1.5367x; 1.3927x over previous
import functools

import jax
import jax.numpy as jnp
from jax import lax
from jax.experimental import pallas as pl
from jax.experimental.pallas import tpu as pltpu
from jax.experimental.pallas import tpu_sc as plsc

F32 = jnp.float32

_NC = 2
_NS = 16
_LANES = 16


def _mlp2_body(x_ref, w1_ref, b1_ref, w2_ref, b2_ref, o_ref):
    h = jnp.maximum(
        jnp.dot(x_ref[...], w1_ref[...], preferred_element_type=F32)
        + b1_ref[...], 0.0)
    o_ref[...] = jnp.dot(h, w2_ref[...], preferred_element_type=F32) + b2_ref[...]


def _mlp2(x, w1, b1, w2, b2, br):
    n, d = x.shape
    h = w1.shape[1]
    dout = w2.shape[1]
    return pl.pallas_call(
        _mlp2_body,
        grid=(n // br,),
        in_specs=[
            pl.BlockSpec((br, d), lambda i: (i, 0)),
            pl.BlockSpec((d, h), lambda i: (0, 0)),
            pl.BlockSpec((1, h), lambda i: (0, 0)),
            pl.BlockSpec((h, dout), lambda i: (0, 0)),
            pl.BlockSpec((1, dout), lambda i: (0, 0)),
        ],
        out_specs=pl.BlockSpec((br, dout), lambda i: (i, 0)),
        out_shape=jax.ShapeDtypeStruct((n, dout), F32),
    )(x, w1, b1.reshape(1, h), w2, b2.reshape(1, dout))


def _srcdst_body(x_ref, ws_ref, wd_ref, xs_ref, xd_ref):
    x = x_ref[...]
    xs_ref[...] = jnp.dot(x, ws_ref[...], preferred_element_type=F32)
    xd_ref[...] = jnp.dot(x, wd_ref[...], preferred_element_type=F32)


def _srcdst(x, ws, wd, br):
    n, h = x.shape
    return pl.pallas_call(
        _srcdst_body,
        grid=(n // br,),
        in_specs=[
            pl.BlockSpec((br, h), lambda i: (i, 0)),
            pl.BlockSpec((h, h), lambda i: (0, 0)),
            pl.BlockSpec((h, h), lambda i: (0, 0)),
        ],
        out_specs=[
            pl.BlockSpec((br, h), lambda i: (i, 0)),
            pl.BlockSpec((br, h), lambda i: (i, 0)),
        ],
        out_shape=[
            jax.ShapeDtypeStruct((n, h), F32),
            jax.ShapeDtypeStruct((n, h), F32),
        ],
    )(x, ws, wd)


def _eterm_body(ef_ref, w_ref, b_ref, o_ref):
    o_ref[...] = (
        jnp.dot(ef_ref[...], w_ref[...], preferred_element_type=F32)
        + b_ref[...])


def _eterm(ef, w, b, be_rows):
    e, de = ef.shape
    h = w.shape[1]
    return pl.pallas_call(
        _eterm_body,
        grid=(e // be_rows,),
        in_specs=[
            pl.BlockSpec((be_rows, de), lambda i: (i, 0)),
            pl.BlockSpec((de, h), lambda i: (0, 0)),
            pl.BlockSpec((1, h), lambda i: (0, 0)),
        ],
        out_specs=pl.BlockSpec((be_rows, h), lambda i: (i, 0)),
        out_shape=jax.ShapeDtypeStruct((e, h), F32),
    )(ef, w, b.reshape(1, h))


def _update_body(x_ref, a0_ref, a1_ref, wn1_ref, wn2_ref, bn_ref, o_ref):
    x = x_ref[...]
    a = a0_ref[...] + a1_ref[...]
    u = (jnp.dot(x, wn1_ref[...], preferred_element_type=F32)
         + jnp.dot(a, wn2_ref[...], preferred_element_type=F32)
         + bn_ref[...])
    o_ref[...] = x + jnp.maximum(u, 0.0)


def _update(x, agg2, wn1, wn2, bn):
    n, h = x.shape
    npad = agg2.shape[0] // 2
    br = 640
    nb2 = npad // br
    return pl.pallas_call(
        _update_body,
        grid=(-(-n // br),),
        in_specs=[
            pl.BlockSpec((br, h), lambda i: (i, 0)),
            pl.BlockSpec((br, h), lambda i: (i, 0)),
            pl.BlockSpec((br, h), lambda i, nb2=nb2: (i + nb2, 0)),
            pl.BlockSpec((h, h), lambda i: (0, 0)),
            pl.BlockSpec((h, h), lambda i: (0, 0)),
            pl.BlockSpec((1, h), lambda i: (0, 0)),
        ],
        out_specs=pl.BlockSpec((br, h), lambda i: (i, 0)),
        out_shape=jax.ShapeDtypeStruct((n, h), F32),
    )(x, agg2, agg2, wn1, wn2, bn.reshape(1, h))


def _make_edge_pass(n, e, h):
    nw = _NC * _NS
    ew = e // nw
    k = 40
    sck = 10
    nsc = ew // (sck * k)
    dr = 32
    npad = ((n + _NS * dr - 1) // (_NS * dr)) * (_NS * dr)
    rpt = npad // _NS
    ndr = rpt // dr
    mesh = plsc.VectorSubcoreMesh(core_axis_name="c", subcore_axis_name="s")

    @functools.partial(
        pl.kernel,
        out_type=jax.ShapeDtypeStruct((2 * npad, h), F32),
        mesh=mesh,
        scratch_types=[
            pltpu.VMEM((sck * k,), jnp.int32),
            pltpu.VMEM((sck * k,), jnp.int32),
            pltpu.VMEM((k, h), F32),
            pltpu.VMEM((k, h), F32),
            pltpu.VMEM((k, h), F32),
            pltpu.VMEM((k, h), F32),
            pltpu.VMEM((k, h), F32),
            pltpu.VMEM((k, h), F32),
            pltpu.VMEM((k, h), F32),
            pltpu.VMEM((dr, h), F32),
            pltpu.VMEM_SHARED((npad, h), F32),
            pltpu.SemaphoreType.DMA,
            pltpu.SemaphoreType.DMA,
            pltpu.SemaphoreType.DMA,
            pltpu.SemaphoreType.DMA,
            pltpu.SemaphoreType.DMA,
            pltpu.SemaphoreType.DMA,
        ],
    )
    def edge_pass(xs_hbm, xd_hbm, et_hbm, src_hbm, dst_hbm, out_hbm,
                  sidx, didx, xsb0, xdb0, etb0, xsb1, xdb1, etb1, mb,
                  stage, agg, g1, g2, g3, g4, g5, g6):
        c = lax.axis_index("c")
        s = lax.axis_index("s")
        wid = s * _NC + c

        sets = [(xsb0, xdb0, etb0, g1, g2, g3),
                (xsb1, xdb1, etb1, g4, g5, g6)]

        def zrow(i, carry):
            for j in range(h // _LANES):
                stage[i, pl.ds(j * _LANES, _LANES)] = jnp.zeros((_LANES,), F32)
            return carry
        lax.fori_loop(0, dr, zrow, 0)
        for q in range(ndr):
            pltpu.sync_copy(stage, agg.at[pl.ds(s * rpt + q * dr, dr)])
        plsc.subcore_barrier()

        def superchunk(si, carry):
            base = wid * ew + si * (sck * k)
            pltpu.sync_copy(src_hbm.at[pl.ds(base, sck * k)], sidx)
            pltpu.sync_copy(dst_hbm.at[pl.ds(base, sck * k)], didx)

            def issue(q):
                xb, db, eb, ga, gb, gc = sets[q % 2]
                ssl = sidx.at[pl.ds(q * k, k)]
                dsl = didx.at[pl.ds(q * k, k)]
                return (pltpu.async_copy(xs_hbm.at[ssl], xb, ga),
                        pltpu.async_copy(xd_hbm.at[dsl], db, gb),
                        pltpu.async_copy(
                            et_hbm.at[pl.ds(base + q * k, k)], eb, gc))

            cps = issue(0)
            for q in range(sck):
                xb, db, eb, _, _, _ = sets[q % 2]
                nxt = issue(q + 1) if q + 1 < sck else None
                for cp in cps:
                    cp.wait()

                def vrow(r, inner, xb=xb, db=db, eb=eb):
                    for j in range(h // _LANES):
                        sl = pl.ds(j * _LANES, _LANES)
                        mb[r, sl] = jnp.maximum(
                            xb[r, sl] + db[r, sl] + eb[r, sl], 0.0)
                    return inner
                lax.fori_loop(0, k, vrow, 0)

                pltpu.sync_copy(mb, agg.at[didx.at[pl.ds(q * k, k)]], add=True)
                cps = nxt
            return carry
        lax.fori_loop(0, nsc, superchunk, 0)
        plsc.subcore_barrier()

        for q in range(ndr):
            pltpu.sync_copy(agg.at[pl.ds(s * rpt + q * dr, dr)], stage)
            pltpu.sync_copy(
                stage, out_hbm.at[pl.ds(c * npad + s * rpt + q * dr, dr)])

    return edge_pass


def kernel(node_features, edge_index, edge_features, enc_W1, enc_b1, enc_W2,
           enc_b2, We, be, Wn, bn, dec_W1, dec_b1, dec_W2, dec_b2):
    n, _ = node_features.shape
    e = edge_index.shape[1]
    h = enc_W1.shape[1]
    nlayers = We.shape[0]

    src = edge_index[0]
    dst = edge_index[1]

    br = 1000
    be_rows = 4000

    x = _mlp2(node_features, enc_W1, enc_b1, enc_W2, enc_b2, br)
    edge_pass = _make_edge_pass(n, e, h)

    for l in range(nlayers):
        ws = We[l, :h]
        wd = We[l, h:2 * h]
        wee = We[l, 2 * h:]
        xs, xd = _srcdst(x, ws, wd, br)
        et = _eterm(edge_features, wee, be[l], be_rows)
        agg2 = edge_pass(xs, xd, et, src, dst)
        x = _update(x, agg2, Wn[l, :h], Wn[l, h:], bn[l])

    return _mlp2(x, dec_W1, dec_b1, dec_W2, dec_b2, br)


# speedup vs baseline: 1.6017x; 1.0423x over previous
import functools

import jax
import jax.numpy as jnp
from jax import lax
from jax.experimental import pallas as pl
from jax.experimental.pallas import tpu as pltpu
from jax.experimental.pallas import tpu_sc as plsc

F32 = jnp.float32

_NC = 2
_NS = 16
_LANES = 16


def _mlp2_body(x_ref, w1_ref, b1_ref, w2_ref, b2_ref, o_ref):
    h = jnp.maximum(
        jnp.dot(x_ref[...], w1_ref[...], preferred_element_type=F32)
        + b1_ref[...], 0.0)
    o_ref[...] = jnp.dot(h, w2_ref[...], preferred_element_type=F32) + b2_ref[...]


def _mlp2(x, w1, b1, w2, b2, br):
    n, d = x.shape
    h = w1.shape[1]
    dout = w2.shape[1]
    return pl.pallas_call(
        _mlp2_body,
        grid=(n // br,),
        in_specs=[
            pl.BlockSpec((br, d), lambda i: (i, 0)),
            pl.BlockSpec((d, h), lambda i: (0, 0)),
            pl.BlockSpec((1, h), lambda i: (0, 0)),
            pl.BlockSpec((h, dout), lambda i: (0, 0)),
            pl.BlockSpec((1, dout), lambda i: (0, 0)),
        ],
        out_specs=pl.BlockSpec((br, dout), lambda i: (i, 0)),
        out_shape=jax.ShapeDtypeStruct((n, dout), F32),
    )(x, w1, b1.reshape(1, h), w2, b2.reshape(1, dout))


def _srcdst_body(x_ref, ws_ref, wd_ref, xs_ref, xd_ref):
    x = x_ref[...]
    xs_ref[...] = jnp.dot(x, ws_ref[...], preferred_element_type=F32)
    xd_ref[...] = jnp.dot(x, wd_ref[...], preferred_element_type=F32)


def _srcdst(x, ws, wd, br):
    n, h = x.shape
    return pl.pallas_call(
        _srcdst_body,
        grid=(n // br,),
        in_specs=[
            pl.BlockSpec((br, h), lambda i: (i, 0)),
            pl.BlockSpec((h, h), lambda i: (0, 0)),
            pl.BlockSpec((h, h), lambda i: (0, 0)),
        ],
        out_specs=[
            pl.BlockSpec((br, h), lambda i: (i, 0)),
            pl.BlockSpec((br, h), lambda i: (i, 0)),
        ],
        out_shape=[
            jax.ShapeDtypeStruct((n, h), F32),
            jax.ShapeDtypeStruct((n, h), F32),
        ],
    )(x, ws, wd)


def _eterm_body(ef_ref, w_ref, b_ref, o_ref):
    o_ref[...] = (
        jnp.dot(ef_ref[...], w_ref[...], preferred_element_type=F32)
        + b_ref[...])


def _eterm(ef, w, b, be_rows):
    e, de = ef.shape
    h = w.shape[1]
    return pl.pallas_call(
        _eterm_body,
        grid=(e // be_rows,),
        in_specs=[
            pl.BlockSpec((be_rows, de), lambda i: (i, 0)),
            pl.BlockSpec((de, h), lambda i: (0, 0)),
            pl.BlockSpec((1, h), lambda i: (0, 0)),
        ],
        out_specs=pl.BlockSpec((be_rows, h), lambda i: (i, 0)),
        out_shape=jax.ShapeDtypeStruct((e, h), F32),
    )(ef, w, b.reshape(1, h))


def _update_body(x_ref, a0_ref, a1_ref, wn1_ref, wn2_ref, bn_ref, o_ref):
    x = x_ref[...]
    a = a0_ref[...] + a1_ref[...]
    u = (jnp.dot(x, wn1_ref[...], preferred_element_type=F32)
         + jnp.dot(a, wn2_ref[...], preferred_element_type=F32)
         + bn_ref[...])
    o_ref[...] = x + jnp.maximum(u, 0.0)


def _update(x, agg2, wn1, wn2, bn):
    n, h = x.shape
    npad = agg2.shape[0] // 2
    br = 640
    nb2 = npad // br
    return pl.pallas_call(
        _update_body,
        grid=(-(-n // br),),
        in_specs=[
            pl.BlockSpec((br, h), lambda i: (i, 0)),
            pl.BlockSpec((br, h), lambda i: (i, 0)),
            pl.BlockSpec((br, h), lambda i, nb2=nb2: (i + nb2, 0)),
            pl.BlockSpec((h, h), lambda i: (0, 0)),
            pl.BlockSpec((h, h), lambda i: (0, 0)),
            pl.BlockSpec((1, h), lambda i: (0, 0)),
        ],
        out_specs=pl.BlockSpec((br, h), lambda i: (i, 0)),
        out_shape=jax.ShapeDtypeStruct((n, h), F32),
    )(x, agg2, agg2, wn1, wn2, bn.reshape(1, h))


def _make_edge_pass(n, e, h):
    nw = _NC * _NS
    ew = e // nw
    k = 40
    sck = 10
    nsc = ew // (sck * k)
    dr = 32
    npad = ((n + _NS * dr - 1) // (_NS * dr)) * (_NS * dr)
    rpt = npad // _NS
    ndr = rpt // dr
    mesh = plsc.VectorSubcoreMesh(core_axis_name="c", subcore_axis_name="s")

    @functools.partial(
        pl.kernel,
        out_type=jax.ShapeDtypeStruct((2 * npad, h), F32),
        mesh=mesh,
        scratch_types=[
            pltpu.VMEM((sck * k,), jnp.int32),
            pltpu.VMEM((sck * k,), jnp.int32),
            pltpu.VMEM((k, h), F32),
            pltpu.VMEM((k, h), F32),
            pltpu.VMEM((k, h), F32),
            pltpu.VMEM((k, h), F32),
            pltpu.VMEM((k, h), F32),
            pltpu.VMEM((k, h), F32),
            pltpu.VMEM((k, h), F32),
            pltpu.VMEM((k, h), F32),
            pltpu.VMEM((dr, h), F32),
            pltpu.VMEM_SHARED((npad, h), F32),
            pltpu.SemaphoreType.DMA,
            pltpu.SemaphoreType.DMA,
            pltpu.SemaphoreType.DMA,
            pltpu.SemaphoreType.DMA,
            pltpu.SemaphoreType.DMA,
            pltpu.SemaphoreType.DMA,
            pltpu.SemaphoreType.DMA,
            pltpu.SemaphoreType.DMA,
        ],
    )
    def edge_pass(xs_hbm, xd_hbm, et_hbm, src_hbm, dst_hbm, out_hbm,
                  sidx, didx, xsb0, xdb0, etb0, xsb1, xdb1, etb1, mb0, mb1,
                  stage, agg, g1, g2, g3, g4, g5, g6, g7, g8):
        c = lax.axis_index("c")
        s = lax.axis_index("s")
        wid = s * _NC + c

        sets = [(xsb0, xdb0, etb0, g1, g2, g3),
                (xsb1, xdb1, etb1, g4, g5, g6)]
        mbs = [(mb0, g7), (mb1, g8)]

        def zrow(i, carry):
            for j in range(h // _LANES):
                stage[i, pl.ds(j * _LANES, _LANES)] = jnp.zeros((_LANES,), F32)
            return carry
        lax.fori_loop(0, dr, zrow, 0)
        for q in range(ndr):
            pltpu.sync_copy(stage, agg.at[pl.ds(s * rpt + q * dr, dr)])
        plsc.subcore_barrier()

        def superchunk(si, carry):
            base = wid * ew + si * (sck * k)
            pltpu.sync_copy(src_hbm.at[pl.ds(base, sck * k)], sidx)
            pltpu.sync_copy(dst_hbm.at[pl.ds(base, sck * k)], didx)

            def issue(q):
                xb, db, eb, ga, gb, gc = sets[q % 2]
                ssl = sidx.at[pl.ds(q * k, k)]
                dsl = didx.at[pl.ds(q * k, k)]
                return (pltpu.async_copy(xs_hbm.at[ssl], xb, ga),
                        pltpu.async_copy(xd_hbm.at[dsl], db, gb),
                        pltpu.async_copy(
                            et_hbm.at[pl.ds(base + q * k, k)], eb, gc))

            cps = issue(0)
            scat = [None, None]
            for q in range(sck):
                xb, db, eb, _, _, _ = sets[q % 2]
                mb, gm = mbs[q % 2]
                nxt = issue(q + 1) if q + 1 < sck else None
                for cp in cps:
                    cp.wait()
                if scat[q % 2] is not None:
                    scat[q % 2].wait()

                def vrow(r, inner, xb=xb, db=db, eb=eb, mb=mb):
                    for j in range(h // _LANES):
                        sl = pl.ds(j * _LANES, _LANES)
                        mb[r, sl] = jnp.maximum(
                            xb[r, sl] + db[r, sl] + eb[r, sl], 0.0)
                    return inner
                lax.fori_loop(0, k, vrow, 0)

                scat[q % 2] = pltpu.async_copy(
                    mb, agg.at[didx.at[pl.ds(q * k, k)]], gm, add=True)
                cps = nxt
            for sp in scat:
                if sp is not None:
                    sp.wait()
            return carry
        lax.fori_loop(0, nsc, superchunk, 0)
        plsc.subcore_barrier()

        for q in range(ndr):
            pltpu.sync_copy(agg.at[pl.ds(s * rpt + q * dr, dr)], stage)
            pltpu.sync_copy(
                stage, out_hbm.at[pl.ds(c * npad + s * rpt + q * dr, dr)])

    return edge_pass


def kernel(node_features, edge_index, edge_features, enc_W1, enc_b1, enc_W2,
           enc_b2, We, be, Wn, bn, dec_W1, dec_b1, dec_W2, dec_b2):
    n, _ = node_features.shape
    e = edge_index.shape[1]
    h = enc_W1.shape[1]
    nlayers = We.shape[0]

    src = edge_index[0]
    dst = edge_index[1]

    br = 1000
    be_rows = 4000

    x = _mlp2(node_features, enc_W1, enc_b1, enc_W2, enc_b2, br)
    edge_pass = _make_edge_pass(n, e, h)

    for l in range(nlayers):
        ws = We[l, :h]
        wd = We[l, h:2 * h]
        wee = We[l, 2 * h:]
        xs, xd = _srcdst(x, ws, wd, br)
        et = _eterm(edge_features, wee, be[l], be_rows)
        agg2 = edge_pass(xs, xd, et, src, dst)
        x = _update(x, agg2, Wn[l, :h], Wn[l, h:], bn[l])

    return _mlp2(x, dec_W1, dec_b1, dec_W2, dec_b2, br)


# speedup vs baseline: 1.6027x; 1.0006x over previous
import functools

import jax
import jax.numpy as jnp
from jax import lax
from jax.experimental import pallas as pl
from jax.experimental.pallas import tpu as pltpu
from jax.experimental.pallas import tpu_sc as plsc

F32 = jnp.float32

_NC = 2
_NS = 16
_LANES = 16


def _mlp2_body(x_ref, w1_ref, b1_ref, w2_ref, b2_ref, o_ref):
    h = jnp.maximum(
        jnp.dot(x_ref[...], w1_ref[...], preferred_element_type=F32)
        + b1_ref[...], 0.0)
    o_ref[...] = jnp.dot(h, w2_ref[...], preferred_element_type=F32) + b2_ref[...]


def _mlp2(x, w1, b1, w2, b2, br):
    n, d = x.shape
    h = w1.shape[1]
    dout = w2.shape[1]
    return pl.pallas_call(
        _mlp2_body,
        grid=(n // br,),
        in_specs=[
            pl.BlockSpec((br, d), lambda i: (i, 0)),
            pl.BlockSpec((d, h), lambda i: (0, 0)),
            pl.BlockSpec((1, h), lambda i: (0, 0)),
            pl.BlockSpec((h, dout), lambda i: (0, 0)),
            pl.BlockSpec((1, dout), lambda i: (0, 0)),
        ],
        out_specs=pl.BlockSpec((br, dout), lambda i: (i, 0)),
        out_shape=jax.ShapeDtypeStruct((n, dout), F32),
    )(x, w1, b1.reshape(1, h), w2, b2.reshape(1, dout))


def _srcdst_body(x_ref, ws_ref, wd_ref, xs_ref, xd_ref):
    x = x_ref[...]
    xs_ref[...] = jnp.dot(x, ws_ref[...], preferred_element_type=F32)
    xd_ref[...] = jnp.dot(x, wd_ref[...], preferred_element_type=F32)


def _srcdst(x, ws, wd, br):
    n, h = x.shape
    return pl.pallas_call(
        _srcdst_body,
        grid=(n // br,),
        in_specs=[
            pl.BlockSpec((br, h), lambda i: (i, 0)),
            pl.BlockSpec((h, h), lambda i: (0, 0)),
            pl.BlockSpec((h, h), lambda i: (0, 0)),
        ],
        out_specs=[
            pl.BlockSpec((br, h), lambda i: (i, 0)),
            pl.BlockSpec((br, h), lambda i: (i, 0)),
        ],
        out_shape=[
            jax.ShapeDtypeStruct((n, h), F32),
            jax.ShapeDtypeStruct((n, h), F32),
        ],
    )(x, ws, wd)


def _eterm_body(ef_ref, w_ref, b_ref, o_ref):
    o_ref[...] = (
        jnp.dot(ef_ref[...], w_ref[...], preferred_element_type=F32)
        + b_ref[...])


def _eterm(ef, w, b, be_rows):
    e, de = ef.shape
    h = w.shape[1]
    return pl.pallas_call(
        _eterm_body,
        grid=(e // be_rows,),
        in_specs=[
            pl.BlockSpec((be_rows, de), lambda i: (i, 0)),
            pl.BlockSpec((de, h), lambda i: (0, 0)),
            pl.BlockSpec((1, h), lambda i: (0, 0)),
        ],
        out_specs=pl.BlockSpec((be_rows, h), lambda i: (i, 0)),
        out_shape=jax.ShapeDtypeStruct((e, h), F32),
    )(ef, w, b.reshape(1, h))


def _update_body(x_ref, a0_ref, a1_ref, wn1_ref, wn2_ref, bn_ref, o_ref):
    x = x_ref[...]
    a = a0_ref[...] + a1_ref[...]
    u = (jnp.dot(x, wn1_ref[...], preferred_element_type=F32)
         + jnp.dot(a, wn2_ref[...], preferred_element_type=F32)
         + bn_ref[...])
    o_ref[...] = x + jnp.maximum(u, 0.0)


def _update(x, agg2, wn1, wn2, bn):
    n, h = x.shape
    npad = agg2.shape[0] // 2
    br = 640
    nb2 = npad // br
    return pl.pallas_call(
        _update_body,
        grid=(-(-n // br),),
        in_specs=[
            pl.BlockSpec((br, h), lambda i: (i, 0)),
            pl.BlockSpec((br, h), lambda i: (i, 0)),
            pl.BlockSpec((br, h), lambda i, nb2=nb2: (i + nb2, 0)),
            pl.BlockSpec((h, h), lambda i: (0, 0)),
            pl.BlockSpec((h, h), lambda i: (0, 0)),
            pl.BlockSpec((1, h), lambda i: (0, 0)),
        ],
        out_specs=pl.BlockSpec((br, h), lambda i: (i, 0)),
        out_shape=jax.ShapeDtypeStruct((n, h), F32),
    )(x, agg2, agg2, wn1, wn2, bn.reshape(1, h))


def _make_edge_pass(n, e, h):
    nw = _NC * _NS
    ew = e // nw
    k = 40
    sck = 10
    nsc = ew // (sck * k)
    dr = 32
    npad = ((n + _NS * dr - 1) // (_NS * dr)) * (_NS * dr)
    rpt = npad // _NS
    ndr = rpt // dr
    mesh = plsc.VectorSubcoreMesh(core_axis_name="c", subcore_axis_name="s")

    @functools.partial(
        pl.kernel,
        out_type=jax.ShapeDtypeStruct((2 * npad, h), F32),
        mesh=mesh,
        scratch_types=[
            pltpu.VMEM((sck * k,), jnp.int32),
            pltpu.VMEM((sck * k,), jnp.int32),
            pltpu.VMEM((k, h), F32),
            pltpu.VMEM((k, h), F32),
            pltpu.VMEM((k, h), F32),
            pltpu.VMEM((k, h), F32),
            pltpu.VMEM((k, h), F32),
            pltpu.VMEM((k, h), F32),
            pltpu.VMEM((k, h), F32),
            pltpu.VMEM((k, h), F32),
            pltpu.VMEM((dr, h), F32),
            pltpu.VMEM_SHARED((npad, h), F32),
            pltpu.SemaphoreType.DMA,
            pltpu.SemaphoreType.DMA,
            pltpu.SemaphoreType.DMA,
            pltpu.SemaphoreType.DMA,
            pltpu.SemaphoreType.DMA,
            pltpu.SemaphoreType.DMA,
            pltpu.SemaphoreType.DMA,
            pltpu.SemaphoreType.DMA,
        ],
    )
    def edge_pass(xs_hbm, xd_hbm, et_hbm, src_hbm, dst_hbm, out_hbm,
                  sidx, didx, xsb0, xdb0, etb0, xsb1, xdb1, etb1, mb0, mb1,
                  stage, agg, g1, g2, g3, g4, g5, g6, g7, g8):
        c = lax.axis_index("c")
        s = lax.axis_index("s")
        wid = s * _NC + c

        sets = [(xsb0, xdb0, etb0, g1, g2, g3),
                (xsb1, xdb1, etb1, g4, g5, g6)]
        mbs = [(mb0, g7), (mb1, g8)]

        def zrow(i, carry):
            for j in range(h // _LANES):
                stage[i, pl.ds(j * _LANES, _LANES)] = jnp.zeros((_LANES,), F32)
            return carry
        lax.fori_loop(0, dr, zrow, 0)
        for q in range(ndr):
            pltpu.sync_copy(stage, agg.at[pl.ds(s * rpt + q * dr, dr)])
        plsc.subcore_barrier()

        def superchunk(si, carry):
            base = wid * ew + si * (sck * k)
            pltpu.sync_copy(src_hbm.at[pl.ds(base, sck * k)], sidx)
            pltpu.sync_copy(dst_hbm.at[pl.ds(base, sck * k)], didx)

            def issue(q):
                xb, db, eb, ga, gb, gc = sets[q % 2]
                ssl = sidx.at[pl.ds(q * k, k)]
                dsl = didx.at[pl.ds(q * k, k)]
                return (pltpu.async_copy(xs_hbm.at[ssl], xb, ga),
                        pltpu.async_copy(xd_hbm.at[dsl], db, gb),
                        pltpu.async_copy(
                            et_hbm.at[pl.ds(base + q * k, k)], eb, gc))

            cps = issue(0)
            scat = [None, None]
            for q in range(sck):
                xb, db, eb, _, _, _ = sets[q % 2]
                mb, gm = mbs[q % 2]
                nxt = issue(q + 1) if q + 1 < sck else None
                for cp in cps:
                    cp.wait()
                if scat[q % 2] is not None:
                    scat[q % 2].wait()

                def vrow(r, inner, xb=xb, db=db, eb=eb, mb=mb):
                    for j in range(h // _LANES):
                        sl = pl.ds(j * _LANES, _LANES)
                        mb[r, sl] = jnp.maximum(
                            xb[r, sl] + db[r, sl] + eb[r, sl], 0.0)
                    return inner
                lax.fori_loop(0, k, vrow, 0)

                scat[q % 2] = pltpu.async_copy(
                    mb, agg.at[didx.at[pl.ds(q * k, k)]], gm, add=True)
                cps = nxt
            for sp in scat:
                if sp is not None:
                    sp.wait()
            return carry
        lax.fori_loop(0, nsc, superchunk, 0)
        plsc.subcore_barrier()

        for q in range(ndr):
            pltpu.sync_copy(agg.at[pl.ds(s * rpt + q * dr, dr)], stage)
            pltpu.sync_copy(
                stage, out_hbm.at[pl.ds(c * npad + s * rpt + q * dr, dr)])

    return edge_pass


def kernel(node_features, edge_index, edge_features, enc_W1, enc_b1, enc_W2,
           enc_b2, We, be, Wn, bn, dec_W1, dec_b1, dec_W2, dec_b2):
    n, _ = node_features.shape
    e = edge_index.shape[1]
    h = enc_W1.shape[1]
    nlayers = We.shape[0]

    src = edge_index[0]
    dst = edge_index[1]

    br = 1000
    be_rows = 4000

    x = _mlp2(node_features, enc_W1, enc_b1, enc_W2, enc_b2, br)
    edge_pass = _make_edge_pass(n, e, h)

    ets = [_eterm(edge_features, We[l, 2 * h:], be[l], be_rows)
           for l in range(nlayers)]

    for l in range(nlayers):
        ws = We[l, :h]
        wd = We[l, h:2 * h]
        xs, xd = _srcdst(x, ws, wd, br)
        agg2 = edge_pass(xs, xd, ets[l], src, dst)
        x = _update(x, agg2, Wn[l, :h], Wn[l, h:], bn[l])

    return _mlp2(x, dec_W1, dec_b1, dec_W2, dec_b2, br)
